# Initial kernel scaffold; baseline (speedup 1.0000x reference)
import jax
import jax.numpy as jnp
from jax.experimental import pallas as pl


def kernel(input_sentence, embedding_table):
    raise NotImplementedError("write your pallas kernel here")


# speedup vs baseline: 6.3404x; 6.3404x over previous
import functools

import jax
import jax.numpy as jnp
from jax import lax
from jax.experimental import pallas as pl
from jax.experimental.pallas import tpu as pltpu
from jax.experimental.pallas import tpu_sc as plsc

D = 128
NW = 32
CHUNK = 128


@functools.lru_cache(maxsize=None)
def _build(B, V):
    b_per_w = B // NW
    n_chunks = b_per_w // CHUNK
    mesh = plsc.VectorSubcoreMesh(core_axis_name="c", subcore_axis_name="s")

    @functools.partial(
        pl.kernel,
        mesh=mesh,
        out_type=jax.ShapeDtypeStruct((B, D), jnp.float32),
        scratch_types=[
            pltpu.VMEM((n_chunks, CHUNK), jnp.int32),
            pltpu.VMEM((CHUNK, D), jnp.float32),
            pltpu.SemaphoreType.DMA,
        ],
    )
    def body(idx_hbm, table_hbm, out_hbm, idx_v, rows_v, gsem):
        cid = lax.axis_index("c")
        sid = lax.axis_index("s")
        wid = sid * 2 + cid
        base = wid * b_per_w
        pltpu.sync_copy(idx_hbm.at[wid], idx_v)

        def chunk_body(j, carry):
            pltpu.async_copy(table_hbm.at[idx_v.at[j]], rows_v, gsem).wait()
            pltpu.sync_copy(rows_v, out_hbm.at[pl.ds(base + j * CHUNK, CHUNK)])
            return carry

        lax.fori_loop(0, n_chunks, chunk_body, 0, unroll=False)

    return body


def kernel(input_sentence, embedding_table):
    batch, hist = input_sentence.shape
    B = batch * hist
    idx = input_sentence.astype(jnp.int32).reshape(NW, (B // NW) // CHUNK, CHUNK)
    out = _build(B, embedding_table.shape[0])(idx, embedding_table)
    return out.reshape(batch, hist, D)


# speedup vs baseline: 9.1814x; 1.4481x over previous
import functools

import jax
import jax.numpy as jnp
from jax import lax
from jax.experimental import pallas as pl
from jax.experimental.pallas import tpu as pltpu
from jax.experimental.pallas import tpu_sc as plsc

D = 128
NW = 32
CHUNK = 128
NBUF = 4
K = 2


@functools.lru_cache(maxsize=None)
def _build(B, V):
    b_per_w = B // NW
    n_chunks = b_per_w // CHUNK
    assert n_chunks % NBUF == 0 and n_chunks >= 2 * NBUF
    mesh = plsc.VectorSubcoreMesh(core_axis_name="c", subcore_axis_name="s")

    @functools.partial(
        pl.kernel,
        mesh=mesh,
        out_type=jax.ShapeDtypeStruct((B, D), jnp.float32),
        scratch_types=[
            pltpu.VMEM((n_chunks, CHUNK), jnp.int32),
            pltpu.VMEM((NBUF, CHUNK, D), jnp.float32),
        ]
        + [pltpu.SemaphoreType.DMA] * (2 * NBUF),
    )
    def body(idx_hbm, table_hbm, out_hbm, idx_v, rows_v, *sems):
        gs, ws = sems[:NBUF], sems[NBUF:]
        cid = lax.axis_index("c")
        sid = lax.axis_index("s")
        wid = sid * 2 + cid
        base = wid * b_per_w
        pltpu.sync_copy(idx_hbm.at[wid], idx_v)

        def gather(c, b):
            return pltpu.make_async_copy(
                table_hbm.at[idx_v.at[c]], rows_v.at[b], gs[b])

        def write(c, b):
            return pltpu.make_async_copy(
                rows_v.at[b], out_hbm.at[pl.ds(base + c * CHUNK, CHUNK)],
                ws[b])

        for c in range(K):
            gather(c, c).start()
        for c in range(K):
            gather(c, c).wait()
            write(c, c).start()
            gather(c + K, (c + K) % NBUF).start()

        def group(g, carry):
            for b_off in range(NBUF):
                c = K + g * NBUF + b_off
                b = (K + b_off) % NBUF
                gather(c, b).wait()
                write(c, b).start()
                nb = (b + K) % NBUF
                write(c - K, nb).wait()
                gather(c + K, nb).start()
            return carry

        lax.fori_loop(0, (n_chunks - 2 * K) // NBUF, group, 0, unroll=False)

        for c in range(n_chunks - K, n_chunks):
            b = c % NBUF
            gather(c, b).wait()
            write(c, b).start()
        for c in range(n_chunks - NBUF, n_chunks):
            write(c, c % NBUF).wait()

    return body


def kernel(input_sentence, embedding_table):
    batch, hist = input_sentence.shape
    B = batch * hist
    idx = input_sentence.astype(jnp.int32).reshape(NW, (B // NW) // CHUNK, CHUNK)
    out = _build(B, embedding_table.shape[0])(idx, embedding_table)
    return out.reshape(batch, hist, D)


# speedup vs baseline: 9.2019x; 1.0022x over previous
import functools

import jax
import jax.numpy as jnp
from jax import lax
from jax.experimental import pallas as pl
from jax.experimental.pallas import tpu as pltpu
from jax.experimental.pallas import tpu_sc as plsc

D = 128
NW = 32
CHUNK = 128
K = 3
NBUF = 2 * K


@functools.lru_cache(maxsize=None)
def _build(B, V):
    b_per_w = B // NW
    n_chunks = b_per_w // CHUNK
    assert n_chunks >= 2 * NBUF
    mesh = plsc.VectorSubcoreMesh(core_axis_name="c", subcore_axis_name="s")

    @functools.partial(
        pl.kernel,
        mesh=mesh,
        out_type=jax.ShapeDtypeStruct((B, D), jnp.float32),
        scratch_types=[
            pltpu.VMEM((n_chunks, CHUNK), jnp.int32),
            pltpu.VMEM((NBUF, CHUNK, D), jnp.float32),
        ]
        + [pltpu.SemaphoreType.DMA] * (2 * NBUF),
    )
    def body(idx_hbm, table_hbm, out_hbm, idx_v, rows_v, *sems):
        gs, ws = sems[:NBUF], sems[NBUF:]
        cid = lax.axis_index("c")
        sid = lax.axis_index("s")
        wid = sid * 2 + cid
        base = wid * b_per_w
        pltpu.sync_copy(idx_hbm.at[wid], idx_v)

        def gather(c, b):
            return pltpu.make_async_copy(
                table_hbm.at[idx_v.at[c]], rows_v.at[b], gs[b])

        def write(c, b):
            return pltpu.make_async_copy(
                rows_v.at[b], out_hbm.at[pl.ds(base + c * CHUNK, CHUNK)],
                ws[b])

        for c in range(K):
            gather(c, c).start()
        for c in range(K):
            gather(c, c).wait()
            write(c, c).start()
            gather(c + K, (c + K) % NBUF).start()

        def slot(c, b):
            gather(c, b).wait()
            write(c, b).start()
            nb = (b + K) % NBUF
            write(c - K, nb).wait()
            gather(c + K, nb).start()

        n_steady = n_chunks - 2 * K
        n_groups, n_rem = divmod(n_steady, NBUF)

        def group(g, carry):
            for b_off in range(NBUF):
                slot(K + g * NBUF + b_off, (K + b_off) % NBUF)
            return carry

        lax.fori_loop(0, n_groups, group, 0, unroll=False)
        for c in range(K + n_groups * NBUF, K + n_steady):
            slot(c, c % NBUF)

        for c in range(n_chunks - K, n_chunks):
            b = c % NBUF
            gather(c, b).wait()
            write(c, b).start()
        for c in range(n_chunks - NBUF, n_chunks):
            write(c, c % NBUF).wait()

    return body


def kernel(input_sentence, embedding_table):
    batch, hist = input_sentence.shape
    B = batch * hist
    idx = input_sentence.astype(jnp.int32).reshape(NW, (B // NW) // CHUNK, CHUNK)
    out = _build(B, embedding_table.shape[0])(idx, embedding_table)
    return out.reshape(batch, hist, D)
